# baseline (speedup 1.0000x reference)
import jax
import jax.numpy as jnp
from jax import lax
from jax.experimental import pallas as pl
from jax.experimental.pallas import tpu as pltpu
from jax.experimental.pallas import tpu_sc as plsc

N_NODES = 10000
N_EDGES = 320000
D = 128

NCORE = 2
NSUB = 16
NW = NCORE * NSUB
CHUNK = 64
NCHUNK = 160
EPW = NCHUNK * CHUNK
E_PAD = NW * EPW
NBUF = 5
ROWS_PER_SUB = 624
TAIL_ROWS = N_NODES - NSUB * ROWS_PER_SUB


def _mul_chunk(rows, w_b):
    def group_body(g, carry):
        w16 = w_b[pl.ds(g * 16, 16)]
        for l in range(16):
            wvec = jnp.full((16,), w16[l], jnp.float32)
            e = g * 16 + l
            for j in range(D // 16):
                sl = pl.ds(j * 16, 16)
                rows[e, sl] = rows[e, sl] * wvec
        return carry

    lax.fori_loop(0, CHUNK // 16, group_body, 0)


def _sc_body(feat_hbm, src_hbm, dst_hbm, ew_hbm, out0_hbm, out1_hbm,
             src_b, dst_b, w_b, rows_v, acc_sh,
             gsems, ssems, csems, dsems, wsems):
    c = lax.axis_index("c")
    s = lax.axis_index("s")
    wid = c * NSUB + s

    def zrow(i, carry):
        for j in range(D // 16):
            rows_v[0][i, pl.ds(j * 16, 16)] = jnp.zeros((16,), jnp.float32)
        return carry

    lax.fori_loop(0, CHUNK, zrow, 0)

    base = s * ROWS_PER_SUB
    nfull = ROWS_PER_SUB // CHUNK
    rem = ROWS_PER_SUB - nfull * CHUNK
    for k in range(nfull):
        pltpu.sync_copy(rows_v[0], acc_sh.at[pl.ds(base + k * CHUNK, CHUNK)])
    if rem:
        pltpu.sync_copy(rows_v[0].at[pl.ds(0, rem)],
                        acc_sh.at[pl.ds(base + nfull * CHUNK, rem)])

    @pl.when(s == NSUB - 1)
    def _zero_tail():
        pltpu.sync_copy(rows_v[0].at[pl.ds(0, TAIL_ROWS)],
                        acc_sh.at[pl.ds(NSUB * ROWS_PER_SUB, TAIL_ROWS)])

    plsc.subcore_barrier()

    with jax.named_scope("sc_prime"):
        for j in range(NBUF):
            pltpu.sync_copy(src_hbm.at[wid, j], src_b[j])
            pltpu.sync_copy(ew_hbm.at[wid, j], w_b[j])
        for j in range(3):
            pltpu.sync_copy(dst_hbm.at[wid, j], dst_b[j])
            pltpu.async_copy(feat_hbm.at[src_b[j]], rows_v[j], gsems[j])

    def chunk_body(q, carry):
        for i in range(NBUF):
            k = q * NBUF + i
            j = i
            jn3 = (i + 3) % NBUF

            pltpu.make_async_copy(
                feat_hbm.at[src_b[j]], rows_v[j], gsems[j]).wait()

            @pl.when(k >= 2)
            def _drain():
                pltpu.make_async_copy(
                    rows_v[jn3], acc_sh.at[dst_b[jn3]], ssems[jn3]).wait()

            @pl.when(k + 3 < NCHUNK)
            def _dst_ahead():
                pltpu.async_copy(
                    dst_hbm.at[wid, k + 3], dst_b[jn3], dsems[jn3])

            @pl.when(k + 3 < NCHUNK)
            def _gather_ahead():
                @pl.when(k >= 2)
                def _wait_src():
                    pltpu.make_async_copy(
                        src_hbm.at[wid, k + 3], src_b[jn3], csems[jn3]).wait()

                pltpu.async_copy(
                    feat_hbm.at[src_b[jn3]], rows_v[jn3], gsems[jn3])

            @pl.when(k + 5 < NCHUNK)
            def _src_ahead():
                pltpu.async_copy(
                    src_hbm.at[wid, k + 5], src_b[j], csems[j])

            @pl.when(k >= 5)
            def _wait_w():
                pltpu.make_async_copy(
                    ew_hbm.at[wid, k], w_b[j], wsems[j]).wait()

            with jax.named_scope("sc_mul"):
                _mul_chunk(rows_v[j], w_b[j])

            @pl.when(k >= 3)
            def _wait_dst():
                pltpu.make_async_copy(
                    dst_hbm.at[wid, k], dst_b[j], dsems[j]).wait()

            pltpu.async_copy(rows_v[j], acc_sh.at[dst_b[j]],
                             ssems[j], add=True)

            @pl.when(k + 5 < NCHUNK)
            def _w_ahead():
                pltpu.async_copy(
                    ew_hbm.at[wid, k + 5], w_b[j], wsems[j])
        return carry

    with jax.named_scope("sc_pipe"):
        lax.fori_loop(0, NCHUNK // NBUF, chunk_body, 0)

    for k in (NCHUNK - 2, NCHUNK - 1):
        j = k % NBUF
        pltpu.make_async_copy(
            rows_v[j], acc_sh.at[dst_b[j]], ssems[j]).wait()
    plsc.subcore_barrier()

    for cc, out_hbm in ((0, out0_hbm), (1, out1_hbm)):
        @pl.when(c == cc)
        def _flush(out_hbm=out_hbm):
            pltpu.sync_copy(acc_sh.at[pl.ds(base, ROWS_PER_SUB)],
                            out_hbm.at[pl.ds(base, ROWS_PER_SUB)])

            @pl.when(s == NSUB - 1)
            def _flush_tail():
                pltpu.sync_copy(
                    acc_sh.at[pl.ds(NSUB * ROWS_PER_SUB, TAIL_ROWS)],
                    out_hbm.at[pl.ds(NSUB * ROWS_PER_SUB, TAIL_ROWS)])


_sc_aggregate = pl.kernel(
    _sc_body,
    out_type=(jax.ShapeDtypeStruct((N_NODES, D), jnp.float32),
              jax.ShapeDtypeStruct((N_NODES, D), jnp.float32)),
    mesh=plsc.VectorSubcoreMesh(core_axis_name="c", subcore_axis_name="s"),
    scratch_types=[
        [pltpu.VMEM((CHUNK,), jnp.int32) for _ in range(NBUF)],
        [pltpu.VMEM((CHUNK,), jnp.int32) for _ in range(NBUF)],
        [pltpu.VMEM((CHUNK,), jnp.float32) for _ in range(NBUF)],
        [pltpu.VMEM((CHUNK, D), jnp.float32) for _ in range(NBUF)],
        pltpu.VMEM_SHARED((N_NODES, D), jnp.float32),
        [pltpu.SemaphoreType.DMA for _ in range(NBUF)],
        [pltpu.SemaphoreType.DMA for _ in range(NBUF)],
        [pltpu.SemaphoreType.DMA for _ in range(NBUF)],
        [pltpu.SemaphoreType.DMA for _ in range(NBUF)],
        [pltpu.SemaphoreType.DMA for _ in range(NBUF)],
    ],
)

ROW_BLK = 1000


def _tc_body(p0_ref, p1_ref, w_ref, o_ref):
    acc = p0_ref[...] + p1_ref[...]
    o_ref[...] = jnp.maximum(
        jnp.dot(acc, w_ref[...], preferred_element_type=jnp.float32), 0.0)


def _tc_finish(p0, p1, W):
    return pl.pallas_call(
        _tc_body,
        grid=(N_NODES // ROW_BLK,),
        in_specs=[
            pl.BlockSpec((ROW_BLK, D), lambda i: (i, 0)),
            pl.BlockSpec((ROW_BLK, D), lambda i: (i, 0)),
            pl.BlockSpec((D, D), lambda i: (0, 0)),
        ],
        out_specs=pl.BlockSpec((ROW_BLK, D), lambda i: (i, 0)),
        out_shape=jax.ShapeDtypeStruct((N_NODES, D), jnp.float32),
    )(p0, p1, W)


@jax.jit
def kernel(feat, edge_index, edge_weight, W):
    pad = E_PAD - N_EDGES
    pad_idx = (jnp.arange(pad, dtype=jnp.int32) * 13) % N_NODES
    src = jnp.concatenate(
        [edge_index[0], pad_idx]).reshape(NW, NCHUNK, CHUNK)
    dst = jnp.concatenate(
        [edge_index[1], pad_idx]).reshape(NW, NCHUNK, CHUNK)
    ew = jnp.concatenate(
        [edge_weight, jnp.zeros((pad,), jnp.float32)]
    ).reshape(NW, NCHUNK, CHUNK)
    p0, p1 = _sc_aggregate(feat, src, dst, ew)
    return _tc_finish(p0, p1, W)


# speedup vs baseline: 1.0754x; 1.0754x over previous
import jax
import jax.numpy as jnp
from jax import lax
from jax.experimental import pallas as pl
from jax.experimental.pallas import tpu as pltpu
from jax.experimental.pallas import tpu_sc as plsc

N_NODES = 10000
N_EDGES = 320000
D = 128

NCORE = 2
NSUB = 16
NW = NCORE * NSUB
EPW = N_EDGES // NW
CHUNK = 64
NCHUNK = 155
TCHUNK = EPW - NCHUNK * CHUNK
TOFF = NCHUNK * CHUNK
NBUF = 5
ROWS_PER_SUB = 624
TAIL_ROWS = N_NODES - NSUB * ROWS_PER_SUB


def _mul_chunk(rows, w_b, ngrp):
    def group_body(g, carry):
        w16 = w_b[pl.ds(g * 16, 16)]
        for l in range(16):
            wvec = jnp.full((16,), w16[l], jnp.float32)
            e = g * 16 + l
            for j in range(D // 16):
                sl = pl.ds(j * 16, 16)
                rows[e, sl] = rows[e, sl] * wvec
        return carry

    lax.fori_loop(0, ngrp, group_body, 0)


def _sc_body(feat_hbm, srci_hbm, dsti_hbm, ew_hbm, out0_hbm, out1_hbm,
             src_b, dst_b, w_b, srct_b, dstt_b, wt_b, rows_v, acc_sh,
             gsems, ssems, csems, dsems, wsems):
    c = lax.axis_index("c")
    s = lax.axis_index("s")
    wid = c * NSUB + s
    ebase = wid * EPW

    def src_hslice(k1):
        return srci_hbm.at[pl.ds(ebase + k1 * CHUNK, CHUNK)]

    def dst_hslice(k1):
        return dsti_hbm.at[pl.ds(ebase + k1 * CHUNK, CHUNK)]

    def w_hslice(k1):
        return ew_hbm.at[pl.ds(ebase + k1 * CHUNK, CHUNK)]

    with jax.named_scope("sc_prime"):
        for j in range(NBUF):
            pltpu.async_copy(src_hslice(j), src_b[j], csems[j])
            pltpu.async_copy(w_hslice(j), w_b[j], wsems[j])
        for j in range(3):
            pltpu.async_copy(dst_hslice(j), dst_b[j], dsems[j])

        def zrow(i, carry):
            for j in range(D // 16):
                rows_v[0][i, pl.ds(j * 16, 16)] = jnp.zeros((16,),
                                                            jnp.float32)
            return carry

        lax.fori_loop(0, CHUNK, zrow, 0)

        base = s * ROWS_PER_SUB
        nfull = ROWS_PER_SUB // CHUNK
        rem = ROWS_PER_SUB - nfull * CHUNK
        for k in range(nfull):
            pltpu.sync_copy(rows_v[0].at[pl.ds(0, CHUNK)],
                            acc_sh.at[pl.ds(base + k * CHUNK, CHUNK)])
        if rem:
            pltpu.sync_copy(rows_v[0].at[pl.ds(0, rem)],
                            acc_sh.at[pl.ds(base + nfull * CHUNK, rem)])

        @pl.when(s == NSUB - 1)
        def _zero_tail():
            pltpu.sync_copy(rows_v[0].at[pl.ds(0, TAIL_ROWS)],
                            acc_sh.at[pl.ds(NSUB * ROWS_PER_SUB, TAIL_ROWS)])

        for j in range(3):
            pltpu.make_async_copy(src_hslice(j), src_b[j], csems[j]).wait()
            pltpu.async_copy(feat_hbm.at[src_b[j]],
                             rows_v[j].at[pl.ds(0, CHUNK)], gsems[j])

    plsc.subcore_barrier()

    def chunk_body(q, carry):
        for i in range(NBUF):
            k = q * NBUF + i
            j = i
            jn3 = (i + 3) % NBUF

            pltpu.make_async_copy(
                feat_hbm.at[src_b[j]], rows_v[j].at[pl.ds(0, CHUNK)],
                gsems[j]).wait()

            @pl.when(k >= 2)
            def _drain():
                pltpu.make_async_copy(
                    rows_v[jn3].at[pl.ds(0, CHUNK)], acc_sh.at[dst_b[jn3]],
                    ssems[jn3]).wait()

            @pl.when(k + 3 < NCHUNK)
            def _dst_ahead():
                pltpu.async_copy(dst_hslice(k + 3), dst_b[jn3], dsems[jn3])

            @pl.when(k + 3 < NCHUNK)
            def _gather_ahead():
                pltpu.make_async_copy(
                    src_hslice(k + 3), src_b[jn3], csems[jn3]).wait()
                pltpu.async_copy(feat_hbm.at[src_b[jn3]],
                                 rows_v[jn3].at[pl.ds(0, CHUNK)], gsems[jn3])

            @pl.when(k + 5 < NCHUNK)
            def _src_ahead():
                pltpu.async_copy(src_hslice(k + 5), src_b[j], csems[j])

            pltpu.make_async_copy(w_hslice(k), w_b[j], wsems[j]).wait()
            with jax.named_scope("sc_mul"):
                _mul_chunk(rows_v[j], w_b[j], CHUNK // 16)

            pltpu.make_async_copy(dst_hslice(k), dst_b[j], dsems[j]).wait()
            pltpu.async_copy(rows_v[j].at[pl.ds(0, CHUNK)],
                             acc_sh.at[dst_b[j]], ssems[j], add=True)

            @pl.when(k + 5 < NCHUNK)
            def _w_ahead():
                pltpu.async_copy(w_hslice(k + 5), w_b[j], wsems[j])
        return carry

    with jax.named_scope("sc_pipe"):
        lax.fori_loop(0, NCHUNK // NBUF, chunk_body, 0)

    with jax.named_scope("sc_tail"):
        pltpu.async_copy(srci_hbm.at[pl.ds(ebase + TOFF, TCHUNK)],
                         srct_b, csems[0])
        pltpu.async_copy(dsti_hbm.at[pl.ds(ebase + TOFF, TCHUNK)],
                         dstt_b, dsems[0])
        pltpu.async_copy(ew_hbm.at[pl.ds(ebase + TOFF, TCHUNK)],
                         wt_b, wsems[0])
        for k in (NCHUNK - 2, NCHUNK - 1):
            j = k % NBUF
            pltpu.make_async_copy(
                rows_v[j].at[pl.ds(0, CHUNK)], acc_sh.at[dst_b[j]],
                ssems[j]).wait()
        pltpu.make_async_copy(srci_hbm.at[pl.ds(ebase + TOFF, TCHUNK)],
                              srct_b, csems[0]).wait()
        pltpu.async_copy(feat_hbm.at[srct_b], rows_v[0], gsems[0])
        pltpu.make_async_copy(feat_hbm.at[srct_b], rows_v[0],
                              gsems[0]).wait()
        pltpu.make_async_copy(ew_hbm.at[pl.ds(ebase + TOFF, TCHUNK)],
                              wt_b, wsems[0]).wait()
        _mul_chunk(rows_v[0], wt_b, TCHUNK // 16)
        pltpu.make_async_copy(dsti_hbm.at[pl.ds(ebase + TOFF, TCHUNK)],
                              dstt_b, dsems[0]).wait()
        pltpu.sync_copy(rows_v[0], acc_sh.at[dstt_b], add=True)

    plsc.subcore_barrier()

    base = s * ROWS_PER_SUB
    for cc, out_hbm in ((0, out0_hbm), (1, out1_hbm)):
        @pl.when(c == cc)
        def _flush(out_hbm=out_hbm):
            pltpu.sync_copy(acc_sh.at[pl.ds(base, ROWS_PER_SUB)],
                            out_hbm.at[pl.ds(base, ROWS_PER_SUB)])

            @pl.when(s == NSUB - 1)
            def _flush_tail():
                pltpu.sync_copy(
                    acc_sh.at[pl.ds(NSUB * ROWS_PER_SUB, TAIL_ROWS)],
                    out_hbm.at[pl.ds(NSUB * ROWS_PER_SUB, TAIL_ROWS)])


_sc_aggregate = pl.kernel(
    _sc_body,
    out_type=(jax.ShapeDtypeStruct((N_NODES, D), jnp.float32),
              jax.ShapeDtypeStruct((N_NODES, D), jnp.float32)),
    mesh=plsc.VectorSubcoreMesh(core_axis_name="c", subcore_axis_name="s"),
    scratch_types=[
        [pltpu.VMEM((CHUNK,), jnp.int32) for _ in range(NBUF)],
        [pltpu.VMEM((CHUNK,), jnp.int32) for _ in range(NBUF)],
        [pltpu.VMEM((CHUNK,), jnp.float32) for _ in range(NBUF)],
        pltpu.VMEM((TCHUNK,), jnp.int32),
        pltpu.VMEM((TCHUNK,), jnp.int32),
        pltpu.VMEM((TCHUNK,), jnp.float32),
        [pltpu.VMEM((TCHUNK, D), jnp.float32)] +
        [pltpu.VMEM((CHUNK, D), jnp.float32) for _ in range(NBUF - 1)],
        pltpu.VMEM_SHARED((N_NODES, D), jnp.float32),
        [pltpu.SemaphoreType.DMA for _ in range(NBUF)],
        [pltpu.SemaphoreType.DMA for _ in range(NBUF)],
        [pltpu.SemaphoreType.DMA for _ in range(NBUF)],
        [pltpu.SemaphoreType.DMA for _ in range(NBUF)],
        [pltpu.SemaphoreType.DMA for _ in range(NBUF)],
    ],
)

ROW_BLK = 1000


def _tc_body(p0_ref, p1_ref, w_ref, o_ref):
    acc = p0_ref[...] + p1_ref[...]
    o_ref[...] = jnp.maximum(
        jnp.dot(acc, w_ref[...], preferred_element_type=jnp.float32), 0.0)


def _tc_finish(p0, p1, W):
    return pl.pallas_call(
        _tc_body,
        grid=(N_NODES // ROW_BLK,),
        in_specs=[
            pl.BlockSpec((ROW_BLK, D), lambda i: (i, 0)),
            pl.BlockSpec((ROW_BLK, D), lambda i: (i, 0)),
            pl.BlockSpec((D, D), lambda i: (0, 0)),
        ],
        out_specs=pl.BlockSpec((ROW_BLK, D), lambda i: (i, 0)),
        out_shape=jax.ShapeDtypeStruct((N_NODES, D), jnp.float32),
    )(p0, p1, W)


@jax.jit
def kernel(feat, edge_index, edge_weight, W):
    p0, p1 = _sc_aggregate(feat, edge_index[0], edge_index[1], edge_weight)
    return _tc_finish(p0, p1, W)


# speedup vs baseline: 1.1497x; 1.0691x over previous
import jax
import jax.numpy as jnp
from jax import lax
from jax.experimental import pallas as pl
from jax.experimental.pallas import tpu as pltpu
from jax.experimental.pallas import tpu_sc as plsc

N_NODES = 10000
N_EDGES = 320000
D = 128

NCORE = 2
NSUB = 16
NW = NCORE * NSUB
EPW = N_EDGES // NW
CHUNK = 64
NCHUNK = 155
TCHUNK = EPW - NCHUNK * CHUNK
TOFF = NCHUNK * CHUNK
NBUF = 5
ROWS_PER_SUB = 624
TAIL_ROWS = N_NODES - NSUB * ROWS_PER_SUB


def _mul_chunk(rows, w_b, ngrp):
    def group_body(g, carry):
        w16 = w_b[pl.ds(g * 16, 16)]
        for l in range(16):
            wvec = jnp.full((16,), w16[l], jnp.float32)
            e = g * 16 + l
            for j in range(D // 16):
                sl = pl.ds(j * 16, 16)
                rows[e, sl] = rows[e, sl] * wvec
        return carry

    lax.fori_loop(0, ngrp, group_body, 0)


def _sc_body(feat_hbm, ei_hbm, ew_hbm, out0_hbm, out1_hbm,
             src_b, dst_b, w_b, srct_b, dstt_b, wt_b, rows_v, acc_sh,
             gsems, ssems, csems, dsems, wsems):
    c = lax.axis_index("c")
    s = lax.axis_index("s")
    wid = c * NSUB + s
    ebase = wid * EPW

    def src_hslice(k1):
        return ei_hbm.at[pl.ds(ebase + k1 * CHUNK, CHUNK)]

    def dst_hslice(k1):
        return ei_hbm.at[pl.ds(N_EDGES + ebase + k1 * CHUNK, CHUNK)]

    def w_hslice(k1):
        return ew_hbm.at[pl.ds(ebase + k1 * CHUNK, CHUNK)]

    with jax.named_scope("sc_prime"):
        for j in range(NBUF):
            pltpu.async_copy(src_hslice(j), src_b[j], csems[j])
            pltpu.async_copy(w_hslice(j), w_b[j], wsems[j])
        for j in range(3):
            pltpu.async_copy(dst_hslice(j), dst_b[j], dsems[j])

        def zrow(i, carry):
            for j in range(D // 16):
                rows_v[0][i, pl.ds(j * 16, 16)] = jnp.zeros((16,),
                                                            jnp.float32)
            return carry

        lax.fori_loop(0, CHUNK, zrow, 0)

        base = s * ROWS_PER_SUB
        nfull = ROWS_PER_SUB // CHUNK
        rem = ROWS_PER_SUB - nfull * CHUNK
        for k in range(nfull):
            pltpu.sync_copy(rows_v[0].at[pl.ds(0, CHUNK)],
                            acc_sh.at[pl.ds(base + k * CHUNK, CHUNK)])
        if rem:
            pltpu.sync_copy(rows_v[0].at[pl.ds(0, rem)],
                            acc_sh.at[pl.ds(base + nfull * CHUNK, rem)])

        @pl.when(s == NSUB - 1)
        def _zero_tail():
            pltpu.sync_copy(rows_v[0].at[pl.ds(0, TAIL_ROWS)],
                            acc_sh.at[pl.ds(NSUB * ROWS_PER_SUB, TAIL_ROWS)])

        for j in range(3):
            pltpu.make_async_copy(src_hslice(j), src_b[j], csems[j]).wait()
            pltpu.async_copy(feat_hbm.at[src_b[j]],
                             rows_v[j].at[pl.ds(0, CHUNK)], gsems[j])

    plsc.subcore_barrier()

    def chunk_body(q, carry):
        for i in range(NBUF):
            k = q * NBUF + i
            j = i
            jn3 = (i + 3) % NBUF

            pltpu.make_async_copy(
                feat_hbm.at[src_b[j]], rows_v[j].at[pl.ds(0, CHUNK)],
                gsems[j]).wait()

            @pl.when(k >= 2)
            def _drain():
                pltpu.make_async_copy(
                    rows_v[jn3].at[pl.ds(0, CHUNK)], acc_sh.at[dst_b[jn3]],
                    ssems[jn3]).wait()

            @pl.when(k + 3 < NCHUNK)
            def _dst_ahead():
                pltpu.async_copy(dst_hslice(k + 3), dst_b[jn3], dsems[jn3])

            @pl.when(k + 3 < NCHUNK)
            def _gather_ahead():
                pltpu.make_async_copy(
                    src_hslice(k + 3), src_b[jn3], csems[jn3]).wait()
                pltpu.async_copy(feat_hbm.at[src_b[jn3]],
                                 rows_v[jn3].at[pl.ds(0, CHUNK)], gsems[jn3])

            @pl.when(k + 5 < NCHUNK)
            def _src_ahead():
                pltpu.async_copy(src_hslice(k + 5), src_b[j], csems[j])

            pltpu.make_async_copy(w_hslice(k), w_b[j], wsems[j]).wait()
            with jax.named_scope("sc_mul"):
                _mul_chunk(rows_v[j], w_b[j], CHUNK // 16)

            pltpu.make_async_copy(dst_hslice(k), dst_b[j], dsems[j]).wait()
            pltpu.async_copy(rows_v[j].at[pl.ds(0, CHUNK)],
                             acc_sh.at[dst_b[j]], ssems[j], add=True)

            @pl.when(k + 5 < NCHUNK)
            def _w_ahead():
                pltpu.async_copy(w_hslice(k + 5), w_b[j], wsems[j])
        return carry

    with jax.named_scope("sc_pipe"):
        lax.fori_loop(0, NCHUNK // NBUF, chunk_body, 0)

    with jax.named_scope("sc_tail"):
        pltpu.async_copy(ei_hbm.at[pl.ds(ebase + TOFF, TCHUNK)],
                         srct_b, csems[0])
        pltpu.async_copy(ei_hbm.at[pl.ds(N_EDGES + ebase + TOFF, TCHUNK)],
                         dstt_b, dsems[0])
        pltpu.async_copy(ew_hbm.at[pl.ds(ebase + TOFF, TCHUNK)],
                         wt_b, wsems[0])
        for k in (NCHUNK - 2, NCHUNK - 1):
            j = k % NBUF
            pltpu.make_async_copy(
                rows_v[j].at[pl.ds(0, CHUNK)], acc_sh.at[dst_b[j]],
                ssems[j]).wait()
        pltpu.make_async_copy(ei_hbm.at[pl.ds(ebase + TOFF, TCHUNK)],
                              srct_b, csems[0]).wait()
        pltpu.async_copy(feat_hbm.at[srct_b], rows_v[0], gsems[0])
        pltpu.make_async_copy(feat_hbm.at[srct_b], rows_v[0],
                              gsems[0]).wait()
        pltpu.make_async_copy(ew_hbm.at[pl.ds(ebase + TOFF, TCHUNK)],
                              wt_b, wsems[0]).wait()
        _mul_chunk(rows_v[0], wt_b, TCHUNK // 16)
        pltpu.make_async_copy(ei_hbm.at[pl.ds(N_EDGES + ebase + TOFF, TCHUNK)],
                              dstt_b, dsems[0]).wait()
        pltpu.sync_copy(rows_v[0], acc_sh.at[dstt_b], add=True)

    plsc.subcore_barrier()

    base = s * ROWS_PER_SUB
    for cc, out_hbm in ((0, out0_hbm), (1, out1_hbm)):
        @pl.when(c == cc)
        def _flush(out_hbm=out_hbm):
            pltpu.sync_copy(acc_sh.at[pl.ds(base, ROWS_PER_SUB)],
                            out_hbm.at[pl.ds(base, ROWS_PER_SUB)])

            @pl.when(s == NSUB - 1)
            def _flush_tail():
                pltpu.sync_copy(
                    acc_sh.at[pl.ds(NSUB * ROWS_PER_SUB, TAIL_ROWS)],
                    out_hbm.at[pl.ds(NSUB * ROWS_PER_SUB, TAIL_ROWS)])


_sc_aggregate = pl.kernel(
    _sc_body,
    out_type=(jax.ShapeDtypeStruct((N_NODES, D), jnp.float32),
              jax.ShapeDtypeStruct((N_NODES, D), jnp.float32)),
    mesh=plsc.VectorSubcoreMesh(core_axis_name="c", subcore_axis_name="s"),
    scratch_types=[
        [pltpu.VMEM((CHUNK,), jnp.int32) for _ in range(NBUF)],
        [pltpu.VMEM((CHUNK,), jnp.int32) for _ in range(NBUF)],
        [pltpu.VMEM((CHUNK,), jnp.float32) for _ in range(NBUF)],
        pltpu.VMEM((TCHUNK,), jnp.int32),
        pltpu.VMEM((TCHUNK,), jnp.int32),
        pltpu.VMEM((TCHUNK,), jnp.float32),
        [pltpu.VMEM((TCHUNK, D), jnp.float32)] +
        [pltpu.VMEM((CHUNK, D), jnp.float32) for _ in range(NBUF - 1)],
        pltpu.VMEM_SHARED((N_NODES, D), jnp.float32),
        [pltpu.SemaphoreType.DMA for _ in range(NBUF)],
        [pltpu.SemaphoreType.DMA for _ in range(NBUF)],
        [pltpu.SemaphoreType.DMA for _ in range(NBUF)],
        [pltpu.SemaphoreType.DMA for _ in range(NBUF)],
        [pltpu.SemaphoreType.DMA for _ in range(NBUF)],
    ],
)

ROW_BLK = 1000


def _tc_body(p0_ref, p1_ref, w_ref, o_ref):
    acc = p0_ref[...] + p1_ref[...]
    o_ref[...] = jnp.maximum(
        jnp.dot(acc, w_ref[...], preferred_element_type=jnp.float32), 0.0)


def _tc_finish(p0, p1, W):
    return pl.pallas_call(
        _tc_body,
        grid=(N_NODES // ROW_BLK,),
        in_specs=[
            pl.BlockSpec((ROW_BLK, D), lambda i: (i, 0)),
            pl.BlockSpec((ROW_BLK, D), lambda i: (i, 0)),
            pl.BlockSpec((D, D), lambda i: (0, 0)),
        ],
        out_specs=pl.BlockSpec((ROW_BLK, D), lambda i: (i, 0)),
        out_shape=jax.ShapeDtypeStruct((N_NODES, D), jnp.float32),
    )(p0, p1, W)


@jax.jit
def kernel(feat, edge_index, edge_weight, W):
    p0, p1 = _sc_aggregate(feat, edge_index.reshape(-1), edge_weight)
    return _tc_finish(p0, p1, W)
